# baseline (speedup 1.0000x reference)
import functools

import jax
import jax.numpy as jnp
from jax import lax
from jax.experimental import pallas as pl
from jax.experimental.pallas import tpu as pltpu
from jax.experimental.pallas import tpu_sc as plsc

N = 10000
E = 320000
IN = 128
H = 256
OUT = 128

NC = 2
NS = 16
CH = 128
EPAD = 327680
NROWS = 10112

_mesh = plsc.VectorSubcoreMesh(core_axis_name="c", subcore_axis_name="s")


def _edge_loop(u_hbm, acc, srcp1, dstp1, sxA, sxB, d0, d1, d2, d3,
               rowsA, rowsB, gsA, gsB, isA, isB, ss0, ss1, ss2, ss3,
               tid, cpt):
    base = tid * cpt

    def ldx(j, sx, dx, isem):
        pltpu.async_copy(srcp1.at[pl.ds((base + j) * CH, CH)], sx, isem)
        pltpu.async_copy(dstp1.at[pl.ds((base + j) * CH, CH)], dx, isem)

    def ldx_wait(j, sx, dx, isem):
        pltpu.make_async_copy(srcp1.at[pl.ds((base + j) * CH, CH)], sx, isem).wait()
        pltpu.make_async_copy(dstp1.at[pl.ds((base + j) * CH, CH)], dx, isem).wait()

    def gath(sx, rows, gsem):
        pltpu.async_copy(u_hbm.at[sx], rows, gsem)

    def gath_wait(sx, rows, gsem):
        pltpu.make_async_copy(u_hbm.at[sx], rows, gsem).wait()

    def scat(rows, dx, ssem):
        pltpu.async_copy(rows, acc.at[dx], ssem, add=True)

    def scat_wait(rows, dx, ssem):
        pltpu.make_async_copy(rows, acc.at[dx], ssem).wait()

    ldx(0, sxA, d0, isA)
    ldx_wait(0, sxA, d0, isA)
    gath(sxA, rowsA, gsA)
    ldx(1, sxB, d1, isB)

    def body(p, carry):
        j0 = 4 * p
        j4 = j0 + 4

        @pl.when(p > 0)
        def _():
            scat_wait(rowsB, d3, ss3)
        ldx_wait(j0 + 1, sxB, d1, isB)
        gath(sxB, rowsB, gsB)
        gath_wait(sxA, rowsA, gsA)
        scat(rowsA, d0, ss0)
        ldx(j0 + 2, sxA, d2, isA)
        gath_wait(sxB, rowsB, gsB)
        scat(rowsB, d1, ss1)
        scat_wait(rowsA, d0, ss0)
        ldx_wait(j0 + 2, sxA, d2, isA)
        gath(sxA, rowsA, gsA)
        ldx(j0 + 3, sxB, d3, isB)
        scat_wait(rowsB, d1, ss1)
        gath_wait(sxA, rowsA, gsA)
        scat(rowsA, d2, ss2)
        ldx_wait(j0 + 3, sxB, d3, isB)
        gath(sxB, rowsB, gsB)

        @pl.when(j4 < cpt)
        def _():
            ldx(j4, sxA, d0, isA)

        scat_wait(rowsA, d2, ss2)
        gath_wait(sxB, rowsB, gsB)
        scat(rowsB, d3, ss3)

        @pl.when(j4 < cpt)
        def _():
            ldx_wait(j4, sxA, d0, isA)
            gath(sxA, rowsA, gsA)

        @pl.when(j4 + 1 < cpt)
        def _():
            ldx(j4 + 1, sxB, d1, isB)

        return carry

    lax.fori_loop(0, cpt // 4, body, 0)
    scat_wait(rowsB, d3, ss3)


def _make_agg(wh):
    rpt = NROWS // NS
    cpt = EPAD // NS // CH
    _scr = (
        [pltpu.VMEM_SHARED((NROWS, wh), jnp.float32)]
        + [pltpu.VMEM((CH,), jnp.int32) for _ in range(6)]
        + [pltpu.VMEM((CH, wh), jnp.float32) for _ in range(2)]
        + [pltpu.SemaphoreType.DMA for _ in range(8)]
    )

    @functools.partial(
        pl.kernel,
        out_type=(
            jax.ShapeDtypeStruct((NROWS, wh), jnp.float32),
            jax.ShapeDtypeStruct((NROWS, wh), jnp.float32),
        ),
        mesh=_mesh,
        scratch_types=_scr,
    )
    def agg(uL, uR, srcp1, dstp1, aggL, aggR, acc, sxA, sxB, d0, d1, d2, d3,
            rowsA, rowsB, gsA, gsB, isA, isB, ss0, ss1, ss2, ss3):
        c = lax.axis_index("c")
        s = lax.axis_index("s")

        def half(u_hbm, out_hbm):
            @pl.when(s == 0)
            def _():
                pltpu.sync_copy(u_hbm, acc.at[pl.ds(0, N)])

            plsc.subcore_barrier()
            _edge_loop(u_hbm, acc, srcp1, dstp1, sxA, sxB, d0, d1, d2, d3,
                       rowsA, rowsB, gsA, gsB, isA, isB, ss0, ss1, ss2, ss3,
                       s, cpt)
            plsc.subcore_barrier()
            pltpu.sync_copy(
                acc.at[pl.ds(s * rpt, rpt)], out_hbm.at[pl.ds(s * rpt, rpt)]
            )

        @pl.when(c == 0)
        def _():
            half(uL, aggL)

        @pl.when(c == 1)
        def _():
            half(uR, aggR)

    return agg


_agg128 = _make_agg(H // 2)

_scr_l0 = (
    [pltpu.VMEM_SHARED((NROWS, IN), jnp.float32)]
    + [pltpu.VMEM((CH,), jnp.int32) for _ in range(6)]
    + [pltpu.VMEM((CH, IN), jnp.float32) for _ in range(2)]
    + [pltpu.SemaphoreType.DMA for _ in range(8)]
)


@functools.partial(
    pl.kernel,
    out_type=jax.ShapeDtypeStruct((NC, NROWS, IN), jnp.float32),
    mesh=_mesh,
    scratch_types=_scr_l0,
)
def _agg_l0(u, seed, srcp1, dstp1, out, acc, sxA, sxB, d0, d1, d2, d3,
            rowsA, rowsB, gsA, gsB, isA, isB, ss0, ss1, ss2, ss3):
    c = lax.axis_index("c")
    s = lax.axis_index("s")
    rpt = NROWS // NS
    cpt = EPAD // (NC * NS) // CH
    w = c * NS + s

    @pl.when(s == 0)
    def _():
        pltpu.sync_copy(seed, acc.at[pl.ds(0, N)])

    plsc.subcore_barrier()
    _edge_loop(u, acc, srcp1, dstp1, sxA, sxB, d0, d1, d2, d3,
               rowsA, rowsB, gsA, gsB, isA, isB, ss0, ss1, ss2, ss3,
               w, cpt)
    plsc.subcore_barrier()
    pltpu.sync_copy(acc.at[pl.ds(s * rpt, rpt)], out.at[c, pl.ds(s * rpt, rpt)])


def _prep_body(deg_ref, x_ref, dis_ref, u_ref, uh_ref):
    deg = deg_ref[:, 0:1] + deg_ref[:, 1:2] + 1.0
    dis = 1.0 / jnp.sqrt(deg)
    dis_ref[...] = dis
    u = x_ref[...] * dis
    u_ref[...] = u
    uh_ref[...] = u * 0.5


def _prep(degt, x):
    return pl.pallas_call(
        _prep_body,
        out_shape=(
            jax.ShapeDtypeStruct((N, 1), jnp.float32),
            jax.ShapeDtypeStruct((N, IN), jnp.float32),
            jax.ShapeDtypeStruct((N, IN), jnp.float32),
        ),
    )(degt, x)


BLK = 1000
NBLK = N // BLK


def _make_mm_body(combine_sum):
    def _mm_body(aggL_ref, aggR_ref, dis_ref, w_ref, b_ref, z_ref,
                 stats_ref, acc_ref):
        i = pl.program_id(0)
        if combine_sum:
            t = (aggL_ref[...] + aggR_ref[...]) * dis_ref[...]
        else:
            t = jnp.concatenate([aggL_ref[...], aggR_ref[...]], axis=1) * dis_ref[...]
        z = jnp.dot(t, w_ref[...], preferred_element_type=jnp.float32) + b_ref[...]
        z_ref[...] = z

        @pl.when(i == 0)
        def _():
            acc_ref[...] = jnp.zeros_like(acc_ref)

        acc_ref[0:1, :] += jnp.sum(z, axis=0, keepdims=True)
        acc_ref[1:2, :] += jnp.sum(z * z, axis=0, keepdims=True)
        stats_ref[...] = acc_ref[...]

    return _mm_body


def _matmul_stats(aggL, aggR, dis, w, b, combine_sum=False):
    wh = aggL.shape[1]
    win = w.shape[0]
    return pl.pallas_call(
        _make_mm_body(combine_sum),
        grid=(NBLK,),
        in_specs=[
            pl.BlockSpec((BLK, wh), lambda i: (i, 0)),
            pl.BlockSpec((BLK, wh), lambda i: (i, 0)),
            pl.BlockSpec((BLK, 1), lambda i: (i, 0)),
            pl.BlockSpec((win, H), lambda i: (0, 0)),
            pl.BlockSpec((1, H), lambda i: (0, 0)),
        ],
        out_specs=(
            pl.BlockSpec((BLK, H), lambda i: (i, 0)),
            pl.BlockSpec((2, H), lambda i: (0, 0)),
        ),
        out_shape=(
            jax.ShapeDtypeStruct((N, H), jnp.float32),
            jax.ShapeDtypeStruct((2, H), jnp.float32),
        ),
        scratch_shapes=[pltpu.VMEM((2, H), jnp.float32)],
    )(aggL, aggR, dis, w, b)


def _var_body(z_ref, stats_ref, vout_ref, acc_ref):
    i = pl.program_id(0)
    mu = stats_ref[0:1, :] * (1.0 / N)
    zc = z_ref[...] - mu

    @pl.when(i == 0)
    def _():
        acc_ref[...] = jnp.zeros_like(acc_ref)

    acc_ref[...] += jnp.sum(zc * zc, axis=0, keepdims=True)
    vout_ref[...] = acc_ref[...]


def _var_pass(z, stats):
    return pl.pallas_call(
        _var_body,
        grid=(NBLK,),
        in_specs=[
            pl.BlockSpec((BLK, H), lambda i: (i, 0)),
            pl.BlockSpec((2, H), lambda i: (0, 0)),
        ],
        out_specs=pl.BlockSpec((1, H), lambda i: (0, 0)),
        out_shape=jax.ShapeDtypeStruct((1, H), jnp.float32),
        scratch_shapes=[pltpu.VMEM((1, H), jnp.float32)],
    )(z, stats)


def _bn_body(z_ref, stats_ref, vsum_ref, g_ref, be_ref, dis_ref, m_ref,
             mo_ref, uL_ref, uR_ref):
    mu = stats_ref[0:1, :] * (1.0 / N)
    var = vsum_ref[...] * (1.0 / N)
    inv = 1.0 / jnp.sqrt(var + 1e-5)
    h = jnp.maximum((z_ref[...] - mu) * (inv * g_ref[...]) + be_ref[...], 0.0)
    mo_ref[...] = jnp.maximum(m_ref[...], h)
    u = h * dis_ref[...]
    uL_ref[...] = u[:, : H // 2]
    uR_ref[...] = u[:, H // 2 :]


def _bn_relu_max(z, stats, vsum, g, be, dis, m):
    return pl.pallas_call(
        _bn_body,
        grid=(NBLK,),
        in_specs=[
            pl.BlockSpec((BLK, H), lambda i: (i, 0)),
            pl.BlockSpec((2, H), lambda i: (0, 0)),
            pl.BlockSpec((1, H), lambda i: (0, 0)),
            pl.BlockSpec((1, H), lambda i: (0, 0)),
            pl.BlockSpec((1, H), lambda i: (0, 0)),
            pl.BlockSpec((BLK, 1), lambda i: (i, 0)),
            pl.BlockSpec((BLK, H), lambda i: (i, 0)),
        ],
        out_specs=(
            pl.BlockSpec((BLK, H), lambda i: (i, 0)),
            pl.BlockSpec((BLK, H // 2), lambda i: (i, 0)),
            pl.BlockSpec((BLK, H // 2), lambda i: (i, 0)),
        ),
        out_shape=(
            jax.ShapeDtypeStruct((N, H), jnp.float32),
            jax.ShapeDtypeStruct((N, H // 2), jnp.float32),
            jax.ShapeDtypeStruct((N, H // 2), jnp.float32),
        ),
    )(z, stats, vsum, g, be, dis, m)


def _head_body(m_ref, w1_ref, b1_ref, w2_ref, b2_ref, o_ref):
    h = jnp.maximum(
        jnp.dot(m_ref[...], w1_ref[...], preferred_element_type=jnp.float32)
        + b1_ref[...],
        0.0,
    )
    o_ref[...] = (
        jnp.dot(h, w2_ref[...], preferred_element_type=jnp.float32) + b2_ref[...]
    )


def _head(m, w1, b1, w2, b2):
    return pl.pallas_call(
        _head_body,
        grid=(NBLK,),
        in_specs=[
            pl.BlockSpec((BLK, H), lambda i: (i, 0)),
            pl.BlockSpec((H, H), lambda i: (0, 0)),
            pl.BlockSpec((1, H), lambda i: (0, 0)),
            pl.BlockSpec((H, OUT), lambda i: (0, 0)),
            pl.BlockSpec((1, OUT), lambda i: (0, 0)),
        ],
        out_specs=pl.BlockSpec((BLK, OUT), lambda i: (i, 0)),
        out_shape=jax.ShapeDtypeStruct((N, OUT), jnp.float32),
    )(m, w1, b1, w2, b2)


def kernel(x, edge_index, W0, b0, g0, be0, W1, b1, g1, be1, W2, b2, g2, be2,
           W3, b3, g3, be3, l1W, l1b, l2W, l2b):
    src = edge_index[0].astype(jnp.int32)
    dst = edge_index[1].astype(jnp.int32)
    pad = EPAD - E
    srcp = jnp.concatenate([src, jnp.zeros((pad,), jnp.int32)])
    dstp = jnp.concatenate([dst, jnp.full((pad,), N, jnp.int32)])

    onesN = jnp.ones((N, IN), jnp.float32)
    halfN = jnp.full((N, IN), 0.5, jnp.float32)
    d2 = _agg_l0(onesN, halfN, srcp, dstp)
    degt = jnp.stack([d2[0, :N, 0], d2[1, :N, 0] - 1.0], axis=1)

    dis, u0, u0h = _prep(degt, x)

    params = [(W0, b0, g0, be0), (W1, b1, g1, be1),
              (W2, b2, g2, be2), (W3, b3, g3, be3)]
    m = jnp.zeros((N, H), jnp.float32)
    uL = uR = None
    for i, (Wl, bl, gl, bel) in enumerate(params):
        if i == 0:
            agg2 = _agg_l0(u0, u0h, srcp, dstp)
            aggL, aggR = agg2[0], agg2[1]
        else:
            aggL, aggR = _agg128(uL, uR, srcp, dstp)
        aggL, aggR = aggL[:N], aggR[:N]
        z, stats = _matmul_stats(
            aggL, aggR, dis, Wl, bl.reshape(1, H), combine_sum=(i == 0)
        )
        vsum = _var_pass(z, stats)
        m, uL, uR = _bn_relu_max(
            z, stats, vsum, gl.reshape(1, H), bel.reshape(1, H), dis, m
        )

    return _head(m, l1W, l1b.reshape(1, H), l2W, l2b.reshape(1, OUT))
